# Initial kernel scaffold; baseline (speedup 1.0000x reference)
import jax
import jax.numpy as jnp
from jax.experimental import pallas as pl


def kernel(codec_input_ids, table, W1, b1, W2, b2):
    raise NotImplementedError("write your pallas kernel here")


# speedup vs baseline: 2.9434x; 2.9434x over previous
import functools

import jax
import jax.numpy as jnp
from jax import lax
from jax.experimental import pallas as pl
from jax.experimental.pallas import tpu as pltpu
from jax.experimental.pallas import tpu_sc as plsc

NUM_CODEBOOKS = 4
CODEBOOK_BITS = 17
CODEBOOK_DIM = 16
HIDDEN_SIZE = 768
B, S = 4, 4096
T = B * S

_NC, _NS = 2, 16
_NW = _NC * _NS
_B_PER_W = T // _NW
_CHUNK = 128
_NCHUNK = _B_PER_W // _CHUNK


@functools.cache
def _gather_sc():
    @functools.partial(
        pl.kernel,
        mesh=plsc.VectorSubcoreMesh(core_axis_name="c", subcore_axis_name="s"),
        compiler_params=pltpu.CompilerParams(use_tc_tiling_on_sc=False),
        out_type=jax.ShapeDtypeStruct((T, CODEBOOK_DIM), jnp.float32),
        scratch_types=[
            pltpu.VMEM((_NCHUNK, _CHUNK), jnp.int32),
            pltpu.VMEM((_B_PER_W, CODEBOOK_DIM), jnp.float32),
            pltpu.SemaphoreType.DMA,
        ],
    )
    def gather(ids_hbm, table_hbm, out_hbm, idx_v, rows_v, sem):
        wid = lax.axis_index("s") * _NC + lax.axis_index("c")
        pltpu.sync_copy(ids_hbm.at[pl.ds(wid * _NCHUNK, _NCHUNK)], idx_v)
        copies = [
            pltpu.async_copy(
                table_hbm.at[idx_v.at[j]],
                rows_v.at[pl.ds(j * _CHUNK, _CHUNK)],
                sem,
            )
            for j in range(_NCHUNK)
        ]
        for cp in copies:
            cp.wait()
        pltpu.sync_copy(rows_v, out_hbm.at[pl.ds(wid * _B_PER_W, _B_PER_W)])

    return gather


_TT = 1024
_NT = T // _TT


def _mlp_body(e_ref, id_ref, w1_ref, b1_ref, w2_ref, b2_ref, o_ref):
    e = e_ref[...]
    ids = id_ref[...]
    c = lax.shift_right_logical(ids, CODEBOOK_BITS)

    eb = e.astype(jnp.bfloat16)
    zero = jnp.zeros_like(eb)
    placed = jnp.concatenate(
        [jnp.where(c == i, eb, zero) for i in range(NUM_CODEBOOKS)], axis=1
    )
    h = lax.dot_general(
        placed, w1_ref[...], (((1,), (0,)), ((), ())),
        preferred_element_type=jnp.float32,
    )

    b1 = b1_ref[...]
    b1_sel = jnp.where(
        c == 0, b1[0:1, :],
        jnp.where(c == 1, b1[1:2, :],
                  jnp.where(c == 2, b1[2:3, :], b1[3:4, :])))
    h = h + b1_sel

    g = 0.5 * h * (1.0 + lax.erf(h * 0.7071067811865476))
    gb = g.astype(jnp.bfloat16)
    gzero = jnp.zeros_like(gb)

    acc = None
    for i in range(NUM_CODEBOOKS):
        gi = jnp.where(c == i, gb, gzero)
        p = lax.dot_general(
            gi, w2_ref[i], (((1,), (0,)), ((), ())),
            preferred_element_type=jnp.float32,
        )
        acc = p if acc is None else acc + p

    b2 = b2_ref[...]
    b2_sel = jnp.where(
        c == 0, b2[0:1, :],
        jnp.where(c == 1, b2[1:2, :],
                  jnp.where(c == 2, b2[2:3, :], b2[3:4, :])))
    o_ref[...] = acc + b2_sel


def _mlp_tc(embeds, ids_col, w1cat, b1, w2, b2):
    return pl.pallas_call(
        _mlp_body,
        grid=(_NT,),
        in_specs=[
            pl.BlockSpec((_TT, CODEBOOK_DIM), lambda i: (i, 0)),
            pl.BlockSpec((_TT, 1), lambda i: (i, 0)),
            pl.BlockSpec((NUM_CODEBOOKS * CODEBOOK_DIM, HIDDEN_SIZE),
                         lambda i: (0, 0)),
            pl.BlockSpec((NUM_CODEBOOKS, HIDDEN_SIZE), lambda i: (0, 0)),
            pl.BlockSpec((NUM_CODEBOOKS, HIDDEN_SIZE, HIDDEN_SIZE),
                         lambda i: (0, 0, 0)),
            pl.BlockSpec((NUM_CODEBOOKS, HIDDEN_SIZE), lambda i: (0, 0)),
        ],
        out_specs=pl.BlockSpec((_TT, HIDDEN_SIZE), lambda i: (i, 0)),
        out_shape=jax.ShapeDtypeStruct((T, HIDDEN_SIZE), jnp.float32),
    )(embeds, ids_col, w1cat, b1, w2, b2)


def kernel(codec_input_ids, table, W1, b1, W2, b2):
    ids = codec_input_ids.reshape(-1).astype(jnp.int32)
    embeds = _gather_sc()(ids.reshape(_NW * _NCHUNK, _CHUNK), table)
    w1cat = W1.reshape(NUM_CODEBOOKS * CODEBOOK_DIM, HIDDEN_SIZE)
    out = _mlp_tc(
        embeds,
        ids.reshape(T, 1),
        w1cat.astype(jnp.bfloat16),
        b1,
        W2.astype(jnp.bfloat16),
        b2,
    )
    return out.reshape(B, S, HIDDEN_SIZE)
